# baseline (speedup 1.0000x reference)
import jax
import jax.numpy as jnp
from jax import lax
from jax.experimental import pallas as pl
from jax.experimental.pallas import tpu as pltpu
from jax.experimental.pallas import tpu_sc as plsc

N_ROWS = 1_000_000
N_COLS = 64
N_OUT = 16_384

ALIGNED = 999_936
COLS_PER_SC = N_COLS // 2
SEG = N_OUT // 16
SLC = 488 * 128
SLC_LAST = ALIGNED - 15 * SLC


def _gather_body(xt, xtail, idxt, ot, col_a, col_b, idx_v, out_v,
                 sem_a, sem_b, isem_a, isem_b, gsem, osem_a, osem_b):
    cid = lax.axis_index("c")
    sid = lax.axis_index("s")
    j0 = cid * COLS_PER_SC

    def stage_halves(col_ref, j):
        h = SLC // 2
        parts = [(sid * SLC, h), (sid * SLC + h, SLC - h)]
        return [(xt.at[j, pl.ds(o, n)], col_ref.at[pl.ds(o, n)]) for o, n in parts]

    def stage_last(col_ref, j):
        h = 246 * 128
        return [(xt.at[j, pl.ds(15 * SLC, h)], col_ref.at[pl.ds(15 * SLC, h)]),
                (xt.at[j, pl.ds(15 * SLC + h, SLC_LAST - h)],
                 col_ref.at[pl.ds(15 * SLC + h, SLC_LAST - h)]),
                (xtail.at[pl.ds(j * 128, 128)], col_ref.at[pl.ds(ALIGNED, 128)])]

    def stage_start(col_ref, j, sem):
        @pl.when(sid < 15)
        def _():
            for s, d in stage_halves(col_ref, j):
                pltpu.make_async_copy(s, d, sem).start()

        @pl.when(sid == 15)
        def _():
            for s, d in stage_last(col_ref, j):
                pltpu.make_async_copy(s, d, sem).start()

    def stage_wait(col_ref, j, sem):
        @pl.when(sid < 15)
        def _():
            for s, d in stage_halves(col_ref, j):
                pltpu.make_async_copy(s, d, sem).wait()

        @pl.when(sid == 15)
        def _():
            for s, d in stage_last(col_ref, j):
                pltpu.make_async_copy(s, d, sem).wait()

    def idx_slot(k):
        return idx_v.at[pl.ds((k % 2) * SEG, SEG)]

    def out_slot(k):
        return out_v.at[pl.ds((k % 2) * SEG, SEG)]

    def idx_start(k):
        pltpu.make_async_copy(idxt.at[j0 + k, pl.ds(sid * SEG, SEG)],
                              idx_slot(k), isem_a if k % 2 == 0 else isem_b).start()

    def idx_wait(k):
        pltpu.make_async_copy(idxt.at[j0 + k, pl.ds(sid * SEG, SEG)],
                              idx_slot(k), isem_a if k % 2 == 0 else isem_b).wait()

    def out_start(k):
        pltpu.make_async_copy(out_slot(k), ot.at[j0 + k, pl.ds(sid * SEG, SEG)],
                              osem_a if k % 2 == 0 else osem_b).start()

    def out_wait(k):
        pltpu.make_async_copy(out_slot(k), ot.at[j0 + k, pl.ds(sid * SEG, SEG)],
                              osem_a if k % 2 == 0 else osem_b).wait()

    stage_start(col_a, j0, sem_a)
    idx_start(0)

    for k in range(COLS_PER_SC):
        j = j0 + k
        buf, sem = (col_a, sem_a) if k % 2 == 0 else (col_b, sem_b)
        if k + 1 < COLS_PER_SC:
            nbuf, nsem = (col_b, sem_b) if k % 2 == 0 else (col_a, sem_a)
            stage_start(nbuf, j + 1, nsem)
            idx_start(k + 1)
        stage_wait(buf, j, sem)
        idx_wait(k)
        if k >= 2:
            out_wait(k - 2)
        plsc.subcore_barrier()

        pltpu.make_async_copy(buf.at[idx_slot(k)], out_slot(k), gsem).start()
        pltpu.make_async_copy(buf.at[idx_slot(k)], out_slot(k), gsem).wait()
        out_start(k)
        plsc.subcore_barrier()

    out_wait(COLS_PER_SC - 2)
    out_wait(COLS_PER_SC - 1)


@jax.jit
def _gather_sc(xt, xtail, idxt):
    mesh = plsc.VectorSubcoreMesh(core_axis_name="c", subcore_axis_name="s")
    return pl.kernel(
        _gather_body,
        out_type=jax.ShapeDtypeStruct((N_COLS, N_OUT), jnp.float32),
        mesh=mesh,
        scratch_types=[
            pltpu.VMEM_SHARED((ALIGNED + 128,), jnp.float32),
            pltpu.VMEM_SHARED((ALIGNED + 128,), jnp.float32),
            pltpu.VMEM((2 * SEG,), jnp.int32),
            pltpu.VMEM((2 * SEG,), jnp.float32),
            pltpu.SemaphoreType.DMA,
            pltpu.SemaphoreType.DMA,
            pltpu.SemaphoreType.DMA,
            pltpu.SemaphoreType.DMA,
            pltpu.SemaphoreType.DMA,
            pltpu.SemaphoreType.DMA,
            pltpu.SemaphoreType.DMA,
        ],
    )(xt, xtail, idxt)


def kernel(x, dim, index, sparse_grad, out):
    xtail = jnp.pad(x[ALIGNED:, :], ((0, 128 - (N_ROWS - ALIGNED)), (0, 0)))
    res_t = _gather_sc(x.T, xtail.T.reshape(-1), index.astype(jnp.int32).T)
    return res_t.T
